# baseline (speedup 1.0000x reference)
import functools

import jax
import jax.numpy as jnp
from jax import lax
from jax.experimental import pallas as pl
from jax.experimental.pallas import tpu as pltpu
from jax.experimental.pallas import tpu_sc as plsc

N = 10000
E = 320000
D = 128
NP = 10240
NC = 2
NS = 16
NW = NC * NS
CE = 128
CH = 80
EW_ = CH * CE
EP = NW * EW_
RPT = NP // NS

_mesh = plsc.VectorSubcoreMesh(core_axis_name="c", subcore_axis_name="s")
_sc_params = pltpu.CompilerParams(needs_layout_passes=False)


@functools.partial(
    pl.kernel,
    out_type=jax.ShapeDtypeStruct((NC, NP), jnp.float32),
    mesh=_mesh,
    scratch_types=[
        pltpu.VMEM((CH, CE), jnp.int32),
        pltpu.VMEM((CH, CE), jnp.float32),
        pltpu.VMEM((RPT,), jnp.float32),
        pltpu.VMEM_SHARED((NP,), jnp.float32),
    ],
    compiler_params=_sc_params,
)
def _deg_kernel(dst_hbm, ew_hbm, out_hbm, dst_v, ew_v, zero_v, acc_s):
  c = lax.axis_index("c")
  s = lax.axis_index("s")
  w = c * NS + s

  def _z(i, carry):
    zero_v[pl.ds(i * 16, 16)] = jnp.zeros((16,), jnp.float32)
    return carry

  lax.fori_loop(0, RPT // 16, _z, 0)
  pltpu.sync_copy(zero_v, acc_s.at[pl.ds(s * RPT, RPT)])
  plsc.subcore_barrier()

  pltpu.sync_copy(dst_hbm.at[w], dst_v)
  pltpu.sync_copy(ew_hbm.at[w], ew_v)

  def _chunk(j, carry):
    pltpu.sync_copy(ew_v.at[j], acc_s.at[dst_v.at[j]], add=True)
    return carry

  lax.fori_loop(0, CH, _chunk, 0)
  plsc.subcore_barrier()

  @pl.when(s == 0)
  def _():
    pltpu.sync_copy(acc_s, out_hbm.at[c])


@functools.partial(
    pl.kernel,
    out_type=jax.ShapeDtypeStruct((NC, NP, D), jnp.float32),
    mesh=_mesh,
    scratch_types=[
        pltpu.VMEM((CH, CE), jnp.int32),
        pltpu.VMEM((CH, CE), jnp.int32),
        pltpu.VMEM((CH, CE), jnp.float32),
        pltpu.VMEM((CE, D), jnp.float32),
        pltpu.VMEM_SHARED((NP, D), jnp.float32),
        pltpu.SemaphoreType.DMA,
    ],
    compiler_params=_sc_params,
)
def _agg_kernel(g_hbm, src_hbm, dst_hbm, ew_hbm, out_hbm,
                src_v, dst_v, ew_v, rows_v, acc_s, gsem):
  c = lax.axis_index("c")
  s = lax.axis_index("s")
  w = c * NS + s

  def _zr(r, carry):
    for k in range(D // 16):
      rows_v[r, pl.ds(k * 16, 16)] = jnp.zeros((16,), jnp.float32)
    return carry

  lax.fori_loop(0, CE, _zr, 0)
  for t in range(RPT // CE):
    pltpu.sync_copy(rows_v, acc_s.at[pl.ds(s * RPT + t * CE, CE)])
  plsc.subcore_barrier()

  pltpu.sync_copy(src_hbm.at[w], src_v)
  pltpu.sync_copy(dst_hbm.at[w], dst_v)
  pltpu.sync_copy(ew_hbm.at[w], ew_v)

  def _chunk(j, carry):
    pltpu.async_copy(g_hbm.at[src_v.at[j]], rows_v, gsem).wait()
    def _row(r, rcarry):
      jv = jnp.full((16,), j, jnp.int32)
      rv = jnp.full((16,), r, jnp.int32)
      ev = plsc.load_gather(ew_v, [jv, rv])
      for k in range(D // 16):
        sl = pl.ds(k * 16, 16)
        rows_v[r, sl] = rows_v[r, sl] * ev
      return rcarry

    lax.fori_loop(0, CE, _row, 0)
    pltpu.sync_copy(rows_v, acc_s.at[dst_v.at[j]], add=True)
    return carry

  lax.fori_loop(0, CH, _chunk, 0)

  plsc.subcore_barrier()
  pltpu.sync_copy(acc_s.at[pl.ds(s * RPT, RPT)],
                  out_hbm.at[c, pl.ds(s * RPT, RPT)])


def _dis_body(degp_ref, dis_ref, dis2_ref):
  d = degp_ref[0] + degp_ref[1] + 1.0
  inv = lax.rsqrt(d)
  dis = jnp.where(d > 0, inv, jnp.float32(0.0))
  dis_ref[...] = dis
  dis2_ref[...] = dis * dis


_dis_call = pl.pallas_call(
    _dis_body,
    out_shape=(
        jax.ShapeDtypeStruct((NP // D, D), jnp.float32),
        jax.ShapeDtypeStruct((NP // D, D), jnp.float32),
    ),
)


def _mm_body(x_ref, w_ref, dis_ref, h_ref, g_ref):
  h = jnp.dot(x_ref[...], w_ref[...], preferred_element_type=jnp.float32)
  h_ref[...] = h
  g_ref[...] = h * dis_ref[...]


_mm_call = pl.pallas_call(
    _mm_body,
    out_shape=(
        jax.ShapeDtypeStruct((NP, D), jnp.float32),
        jax.ShapeDtypeStruct((NP, D), jnp.float32),
    ),
)


def _comb_body(p_ref, h_ref, dis_ref, dis2_ref, b_ref, z_ref):
  m = (p_ref[0] + p_ref[1]) * dis_ref[...] + h_ref[...] * dis2_ref[...] + b_ref[...]
  z_ref[...] = jnp.maximum(m, 0.0)


_comb_call = pl.pallas_call(
    _comb_body,
    out_shape=jax.ShapeDtypeStruct((NP, D), jnp.float32),
)


def kernel(input, adj, adj_wts, W1, b1, W2, b2):
  x = jnp.pad(input, ((0, NP - N), (0, 0)))
  src = adj[0].astype(jnp.int32)
  dst = adj[1].astype(jnp.int32)
  ew = adj_wts.astype(jnp.float32)
  pad = EP - E
  src_p = jnp.concatenate([src, jnp.zeros((pad,), jnp.int32)]).reshape(NW, CH, CE)
  dst_p = jnp.concatenate([dst, jnp.zeros((pad,), jnp.int32)]).reshape(NW, CH, CE)
  ew_p = jnp.concatenate([ew, jnp.zeros((pad,), jnp.float32)]).reshape(NW, CH, CE)

  degp = _deg_kernel(dst_p, ew_p)
  dis80, dis280 = _dis_call(degp.reshape(NC, NP // D, D))
  dis_col = dis80.reshape(NP, 1)
  dis2_col = dis280.reshape(NP, 1)

  h1, g1 = _mm_call(x, W1, dis_col)
  p1 = _agg_kernel(g1, src_p, dst_p, ew_p)
  z1 = _comb_call(p1, h1, dis_col, dis2_col, b1.reshape(1, D))
  h2, g2 = _mm_call(z1, W2, dis_col)
  p2 = _agg_kernel(g2, src_p, dst_p, ew_p)
  z2 = _comb_call(p2, h2, dis_col, dis2_col, b2.reshape(1, D))
  return z2[:N]


# speedup vs baseline: 2.1411x; 2.1411x over previous
import functools

import jax
import jax.numpy as jnp
from jax import lax
from jax.experimental import pallas as pl
from jax.experimental.pallas import tpu as pltpu
from jax.experimental.pallas import tpu_sc as plsc

N = 10000
E = 320000
D = 128
NP = 10240
NC = 2
NS = 16
NW = NC * NS
CE = 128
CH = 80
EW_ = CH * CE
EP = NW * EW_
RPT = NP // NS

_mesh = plsc.VectorSubcoreMesh(core_axis_name="c", subcore_axis_name="s")
_sc_params = pltpu.CompilerParams(needs_layout_passes=False)


@functools.partial(
    pl.kernel,
    out_type=jax.ShapeDtypeStruct((NC, NP), jnp.float32),
    mesh=_mesh,
    scratch_types=[
        pltpu.VMEM((CH, CE), jnp.int32),
        pltpu.VMEM((CH, CE), jnp.float32),
        pltpu.VMEM((RPT,), jnp.float32),
        pltpu.VMEM_SHARED((NP,), jnp.float32),
    ],
    compiler_params=_sc_params,
)
def _deg_kernel(dst_hbm, ew_hbm, out_hbm, dst_v, ew_v, zero_v, acc_s):
  c = lax.axis_index("c")
  s = lax.axis_index("s")
  w = c * NS + s

  def _z(i, carry):
    zero_v[pl.ds(i * 16, 16)] = jnp.zeros((16,), jnp.float32)
    return carry

  lax.fori_loop(0, RPT // 16, _z, 0)
  pltpu.sync_copy(zero_v, acc_s.at[pl.ds(s * RPT, RPT)])
  plsc.subcore_barrier()

  pltpu.sync_copy(dst_hbm.at[w], dst_v)
  pltpu.sync_copy(ew_hbm.at[w], ew_v)

  def _chunk(j, carry):
    pltpu.sync_copy(ew_v.at[j], acc_s.at[dst_v.at[j]], add=True)
    return carry

  lax.fori_loop(0, CH, _chunk, 0)
  plsc.subcore_barrier()

  @pl.when(s == 0)
  def _():
    pltpu.sync_copy(acc_s, out_hbm.at[c])


@functools.partial(
    pl.kernel,
    out_type=jax.ShapeDtypeStruct((NC, NP, D), jnp.float32),
    mesh=_mesh,
    scratch_types=[
        pltpu.VMEM((CH, CE), jnp.int32),
        pltpu.VMEM((CH, CE), jnp.int32),
        pltpu.VMEM((CH, CE), jnp.float32),
        pltpu.VMEM((CE, D), jnp.float32),
        pltpu.VMEM_SHARED((NP, D), jnp.float32),
        pltpu.SemaphoreType.DMA,
    ],
    compiler_params=_sc_params,
)
def _agg_kernel(g_hbm, src_hbm, dst_hbm, ew_hbm, out_hbm,
                src_v, dst_v, ew_v, rows_v, acc_s, gsem):
  c = lax.axis_index("c")
  s = lax.axis_index("s")
  w = c * NS + s

  def _zr(r, carry):
    for k in range(D // 16):
      rows_v[r, pl.ds(k * 16, 16)] = jnp.zeros((16,), jnp.float32)
    return carry

  lax.fori_loop(0, CE, _zr, 0)
  for t in range(RPT // CE):
    pltpu.sync_copy(rows_v, acc_s.at[pl.ds(s * RPT + t * CE, CE)])
  plsc.subcore_barrier()

  pltpu.sync_copy(src_hbm.at[w], src_v)
  pltpu.sync_copy(dst_hbm.at[w], dst_v)
  pltpu.sync_copy(ew_hbm.at[w], ew_v)

  def _chunk(j, carry):
    pltpu.async_copy(g_hbm.at[src_v.at[j]], rows_v, gsem).wait()
    def _row(r, rcarry):
      jv = jnp.full((16,), j, jnp.int32)
      rv = jnp.full((16,), r, jnp.int32)
      ev = plsc.load_gather(ew_v, [jv, rv])
      for k in range(D // 16):
        sl = pl.ds(k * 16, 16)
        rows_v[r, sl] = rows_v[r, sl] * ev
      return rcarry

    lax.fori_loop(0, CE, _row, 0)
    pltpu.sync_copy(rows_v, acc_s.at[dst_v.at[j]], add=True)
    return carry

  lax.fori_loop(0, CH, _chunk, 0)

  plsc.subcore_barrier()
  pltpu.sync_copy(acc_s.at[pl.ds(s * RPT, RPT)],
                  out_hbm.at[c, pl.ds(s * RPT, RPT)])


def _dis_body(degp_ref, dis_ref, dis2_ref):
  d = degp_ref[0] + degp_ref[1] + 1.0
  inv = lax.rsqrt(d)
  dis = jnp.where(d > 0, inv, jnp.float32(0.0))
  dis_ref[...] = dis
  dis2_ref[...] = dis * dis


_dis_call = pl.pallas_call(
    _dis_body,
    out_shape=(
        jax.ShapeDtypeStruct((NP // D, D), jnp.float32),
        jax.ShapeDtypeStruct((NP // D, D), jnp.float32),
    ),
)


def _mm_body(x_ref, w_ref, dis_ref, h_ref, g_ref):
  h = jnp.dot(x_ref[...], w_ref[...], preferred_element_type=jnp.float32)
  h_ref[...] = h
  g_ref[...] = h * dis_ref[...]


_mm_call = pl.pallas_call(
    _mm_body,
    out_shape=(
        jax.ShapeDtypeStruct((NP, D), jnp.float32),
        jax.ShapeDtypeStruct((NP, D), jnp.float32),
    ),
)


def _comb_body(p_ref, h_ref, dis_ref, dis2_ref, b_ref, z_ref):
  m = (p_ref[0] + p_ref[1]) * dis_ref[...] + h_ref[...] * dis2_ref[...] + b_ref[...]
  z_ref[...] = jnp.maximum(m, 0.0)


_comb_call = pl.pallas_call(
    _comb_body,
    out_shape=jax.ShapeDtypeStruct((NP, D), jnp.float32),
)


def kernel(input, adj, adj_wts, W1, b1, W2, b2):
  x = jnp.pad(input, ((0, NP - N), (0, 0)))
  src = adj[0].astype(jnp.int32)
  dst = adj[1].astype(jnp.int32)
  ew = adj_wts.astype(jnp.float32)
  pad = EP - E
  spread = (jnp.arange(pad, dtype=jnp.int32) * 8) % N
  src_p = jnp.concatenate([src, spread]).reshape(NW, CH, CE)
  dst_p = jnp.concatenate([dst, spread]).reshape(NW, CH, CE)
  ew_p = jnp.concatenate([ew, jnp.zeros((pad,), jnp.float32)]).reshape(NW, CH, CE)

  degp = _deg_kernel(dst_p, ew_p)
  dis80, dis280 = _dis_call(degp.reshape(NC, NP // D, D))
  dis_col = dis80.reshape(NP, 1)
  dis2_col = dis280.reshape(NP, 1)

  h1, g1 = _mm_call(x, W1, dis_col)
  p1 = _agg_kernel(g1, src_p, dst_p, ew_p)
  z1 = _comb_call(p1, h1, dis_col, dis2_col, b1.reshape(1, D))
  h2, g2 = _mm_call(z1, W2, dis_col)
  p2 = _agg_kernel(g2, src_p, dst_p, ew_p)
  z2 = _comb_call(p2, h2, dis_col, dis2_col, b2.reshape(1, D))
  return z2[:N]


# speedup vs baseline: 3.0199x; 1.4104x over previous
import functools

import jax
import jax.numpy as jnp
from jax import lax
from jax.experimental import pallas as pl
from jax.experimental.pallas import tpu as pltpu
from jax.experimental.pallas import tpu_sc as plsc

N = 10000
E = 320000
D = 128
NP = 10240
NC = 2
NS = 16
NW = NC * NS
CE = 128
CH = 80
GRP = 16
EW_ = CH * CE
EP = NW * EW_
RPT = NP // NS

_mesh = plsc.VectorSubcoreMesh(core_axis_name="c", subcore_axis_name="s")
_sc_params = pltpu.CompilerParams(needs_layout_passes=False)


@functools.partial(
    pl.kernel,
    out_type=jax.ShapeDtypeStruct((NC, NP), jnp.float32),
    mesh=_mesh,
    scratch_types=[
        pltpu.VMEM((CH, CE), jnp.int32),
        pltpu.VMEM((CH, CE), jnp.float32),
        pltpu.VMEM((RPT,), jnp.float32),
        pltpu.VMEM_SHARED((NP,), jnp.float32),
    ],
    compiler_params=_sc_params,
)
def _deg_kernel(dst_hbm, ew_hbm, out_hbm, dst_v, ew_v, zero_v, acc_s):
  c = lax.axis_index("c")
  s = lax.axis_index("s")
  w = c * NS + s

  def _z(i, carry):
    zero_v[pl.ds(i * 16, 16)] = jnp.zeros((16,), jnp.float32)
    return carry

  lax.fori_loop(0, RPT // 16, _z, 0)
  pltpu.sync_copy(zero_v, acc_s.at[pl.ds(s * RPT, RPT)])
  plsc.subcore_barrier()

  pltpu.sync_copy(dst_hbm.at[w], dst_v)
  pltpu.sync_copy(ew_hbm.at[w], ew_v)

  def _chunk(j, carry):
    pltpu.sync_copy(ew_v.at[j], acc_s.at[dst_v.at[j]], add=True)
    return carry

  lax.fori_loop(0, CH, _chunk, 0)
  plsc.subcore_barrier()

  @pl.when(s == 0)
  def _():
    pltpu.sync_copy(acc_s, out_hbm.at[c])


@functools.partial(
    pl.kernel,
    out_type=jax.ShapeDtypeStruct((NC, NP, D), jnp.float32),
    mesh=_mesh,
    scratch_types=[
        pltpu.VMEM((GRP, CE), jnp.int32),
        pltpu.VMEM((GRP, CE), jnp.int32),
        pltpu.VMEM((GRP, CE), jnp.float32),
        pltpu.VMEM((CE, D), jnp.float32),
        pltpu.VMEM((CE, D), jnp.float32),
        pltpu.VMEM_SHARED((NP, D), jnp.float32),
        pltpu.SemaphoreType.DMA,
        pltpu.SemaphoreType.DMA,
    ],
    compiler_params=_sc_params,
)
def _agg_kernel(g_hbm, src_hbm, dst_hbm, ew_hbm, out_hbm,
                src_v, dst_v, ew_v, rows_a, rows_b, acc_s, gsem_a, gsem_b):
  c = lax.axis_index("c")
  s = lax.axis_index("s")
  w = c * NS + s

  def _zr(r, carry):
    for k in range(D // 16):
      rows_a[r, pl.ds(k * 16, 16)] = jnp.zeros((16,), jnp.float32)
    return carry

  lax.fori_loop(0, CE, _zr, 0)
  for t in range(RPT // CE):
    pltpu.sync_copy(rows_a, acc_s.at[pl.ds(s * RPT + t * CE, CE)])
  plsc.subcore_barrier()

  def _scale_scatter(u, rows):
    jv = jnp.full((16,), u, jnp.int32)

    def _rowgrp(g, rcarry):
      for q in range(4):
        r = g * 4 + q
        rv = jnp.full((16,), r, jnp.int32)
        ev = plsc.load_gather(ew_v, [jv, rv])
        for k in range(D // 16):
          sl = pl.ds(k * 16, 16)
          rows[r, sl] = rows[r, sl] * ev
      return rcarry

    lax.fori_loop(0, CE // 4, _rowgrp, 0)
    pltpu.sync_copy(rows, acc_s.at[dst_v.at[u]], add=True)

  bufs = ((rows_a, gsem_a), (rows_b, gsem_b))

  def _group(g, carry):
    j0 = GRP * g
    pltpu.sync_copy(src_hbm.at[w, pl.ds(j0, GRP)], src_v)
    pltpu.sync_copy(dst_hbm.at[w, pl.ds(j0, GRP)], dst_v)
    pltpu.sync_copy(ew_hbm.at[w, pl.ds(j0, GRP)], ew_v)
    pltpu.async_copy(g_hbm.at[src_v.at[0]], rows_a, gsem_a)
    for u in range(GRP):
      rows, gsem = bufs[u % 2]
      nrows, ngsem = bufs[(u + 1) % 2]
      if u + 1 < GRP:
        pltpu.async_copy(g_hbm.at[src_v.at[u + 1]], nrows, ngsem)
      pltpu.make_async_copy(g_hbm.at[src_v.at[u]], rows, gsem).wait()
      _scale_scatter(u, rows)
    return carry

  lax.fori_loop(0, CH // GRP, _group, 0)

  plsc.subcore_barrier()
  pltpu.sync_copy(acc_s.at[pl.ds(s * RPT, RPT)],
                  out_hbm.at[c, pl.ds(s * RPT, RPT)])


def _dis_body(degp_ref, dis_ref, dis2_ref):
  d = degp_ref[0] + degp_ref[1] + 1.0
  inv = lax.rsqrt(d)
  dis = jnp.where(d > 0, inv, jnp.float32(0.0))
  dis_ref[...] = dis
  dis2_ref[...] = dis * dis


_dis_call = pl.pallas_call(
    _dis_body,
    out_shape=(
        jax.ShapeDtypeStruct((NP // D, D), jnp.float32),
        jax.ShapeDtypeStruct((NP // D, D), jnp.float32),
    ),
)


def _mm_body(x_ref, w_ref, dis_ref, h_ref, g_ref):
  h = jnp.dot(x_ref[...], w_ref[...], preferred_element_type=jnp.float32)
  h_ref[...] = h
  g_ref[...] = h * dis_ref[...]


_mm_call = pl.pallas_call(
    _mm_body,
    out_shape=(
        jax.ShapeDtypeStruct((NP, D), jnp.float32),
        jax.ShapeDtypeStruct((NP, D), jnp.float32),
    ),
)


def _comb_body(p_ref, h_ref, dis_ref, dis2_ref, b_ref, z_ref):
  m = (p_ref[0] + p_ref[1]) * dis_ref[...] + h_ref[...] * dis2_ref[...] + b_ref[...]
  z_ref[...] = jnp.maximum(m, 0.0)


_comb_call = pl.pallas_call(
    _comb_body,
    out_shape=jax.ShapeDtypeStruct((NP, D), jnp.float32),
)


def kernel(input, adj, adj_wts, W1, b1, W2, b2):
  x = jnp.pad(input, ((0, NP - N), (0, 0)))
  src = adj[0].astype(jnp.int32)
  dst = adj[1].astype(jnp.int32)
  ew = adj_wts.astype(jnp.float32)
  pad = EP - E
  spread = (jnp.arange(pad, dtype=jnp.int32) * 8) % N
  src_p = jnp.concatenate([src, spread]).reshape(NW, CH, CE)
  dst_p = jnp.concatenate([dst, spread]).reshape(NW, CH, CE)
  ew_p = jnp.concatenate([ew, jnp.zeros((pad,), jnp.float32)]).reshape(NW, CH, CE)

  degp = _deg_kernel(dst_p, ew_p)
  dis80, dis280 = _dis_call(degp.reshape(NC, NP // D, D))
  dis_col = dis80.reshape(NP, 1)
  dis2_col = dis280.reshape(NP, 1)

  h1, g1 = _mm_call(x, W1, dis_col)
  p1 = _agg_kernel(g1, src_p, dst_p, ew_p)
  z1 = _comb_call(p1, h1, dis_col, dis2_col, b1.reshape(1, D))
  h2, g2 = _mm_call(z1, W2, dis_col)
  p2 = _agg_kernel(g2, src_p, dst_p, ew_p)
  z2 = _comb_call(p2, h2, dis_col, dis2_col, b2.reshape(1, D))
  return z2[:N]


# speedup vs baseline: 3.1390x; 1.0394x over previous
import functools

import jax
import jax.numpy as jnp
from jax import lax
from jax.experimental import pallas as pl
from jax.experimental.pallas import tpu as pltpu
from jax.experimental.pallas import tpu_sc as plsc

N = 10000
E = 320000
D = 128
NP = 10240
NC = 2
NS = 16
NW = NC * NS
CE = 128
CH = 80
GRP = 40
EW_ = CH * CE
EP = NW * EW_
RPT = NP // NS

_mesh = plsc.VectorSubcoreMesh(core_axis_name="c", subcore_axis_name="s")
_sc_params = pltpu.CompilerParams(needs_layout_passes=False)


@functools.partial(
    pl.kernel,
    out_type=jax.ShapeDtypeStruct((NC, NP), jnp.float32),
    mesh=_mesh,
    scratch_types=[
        pltpu.VMEM((CH, CE), jnp.int32),
        pltpu.VMEM((CH, CE), jnp.float32),
        pltpu.VMEM((RPT,), jnp.float32),
        pltpu.VMEM_SHARED((NP,), jnp.float32),
    ],
    compiler_params=_sc_params,
)
def _deg_kernel(dst_hbm, ew_hbm, out_hbm, dst_v, ew_v, zero_v, acc_s):
  c = lax.axis_index("c")
  s = lax.axis_index("s")
  w = c * NS + s

  def _z(i, carry):
    zero_v[pl.ds(i * 16, 16)] = jnp.zeros((16,), jnp.float32)
    return carry

  lax.fori_loop(0, RPT // 16, _z, 0)
  pltpu.sync_copy(zero_v, acc_s.at[pl.ds(s * RPT, RPT)])
  plsc.subcore_barrier()

  pltpu.sync_copy(dst_hbm.at[w], dst_v)
  pltpu.sync_copy(ew_hbm.at[w], ew_v)

  def _chunk(j, carry):
    pltpu.sync_copy(ew_v.at[j], acc_s.at[dst_v.at[j]], add=True)
    return carry

  lax.fori_loop(0, CH, _chunk, 0)
  plsc.subcore_barrier()

  @pl.when(s == 0)
  def _():
    pltpu.sync_copy(acc_s, out_hbm.at[c])


@functools.partial(
    pl.kernel,
    out_type=jax.ShapeDtypeStruct((NC, NP, D), jnp.float32),
    mesh=_mesh,
    scratch_types=[
        pltpu.VMEM((GRP, CE), jnp.int32),
        pltpu.VMEM((GRP, CE), jnp.int32),
        pltpu.VMEM((GRP, CE), jnp.float32),
        pltpu.VMEM((CE, D), jnp.float32),
        pltpu.VMEM((CE, D), jnp.float32),
        pltpu.VMEM_SHARED((NP, D), jnp.float32),
        pltpu.SemaphoreType.DMA,
        pltpu.SemaphoreType.DMA,
    ],
    compiler_params=_sc_params,
)
def _agg_kernel(g_hbm, src_hbm, dst_hbm, ew_hbm, out_hbm,
                src_v, dst_v, ew_v, rows_a, rows_b, acc_s, gsem_a, gsem_b):
  c = lax.axis_index("c")
  s = lax.axis_index("s")
  w = c * NS + s

  def _zr(r, carry):
    for k in range(D // 16):
      rows_a[r, pl.ds(k * 16, 16)] = jnp.zeros((16,), jnp.float32)
    return carry

  lax.fori_loop(0, CE, _zr, 0)
  for t in range(RPT // CE):
    pltpu.sync_copy(rows_a, acc_s.at[pl.ds(s * RPT + t * CE, CE)])
  plsc.subcore_barrier()

  def _scale_scatter(u, rows):
    jv = jnp.full((16,), u, jnp.int32)

    def _rowgrp(g, rcarry):
      for q in range(4):
        r = g * 4 + q
        rv = jnp.full((16,), r, jnp.int32)
        ev = plsc.load_gather(ew_v, [jv, rv])
        for k in range(D // 16):
          sl = pl.ds(k * 16, 16)
          rows[r, sl] = rows[r, sl] * ev
      return rcarry

    lax.fori_loop(0, CE // 4, _rowgrp, 0)
    pltpu.sync_copy(rows, acc_s.at[dst_v.at[u]], add=True)

  bufs = ((rows_a, gsem_a), (rows_b, gsem_b))

  def _group(g, carry):
    j0 = GRP * g
    pltpu.sync_copy(src_hbm.at[w, pl.ds(j0, GRP)], src_v)
    pltpu.sync_copy(dst_hbm.at[w, pl.ds(j0, GRP)], dst_v)
    pltpu.sync_copy(ew_hbm.at[w, pl.ds(j0, GRP)], ew_v)
    pltpu.async_copy(g_hbm.at[src_v.at[0]], rows_a, gsem_a)
    for u in range(GRP):
      rows, gsem = bufs[u % 2]
      nrows, ngsem = bufs[(u + 1) % 2]
      if u + 1 < GRP:
        pltpu.async_copy(g_hbm.at[src_v.at[u + 1]], nrows, ngsem)
      pltpu.make_async_copy(g_hbm.at[src_v.at[u]], rows, gsem).wait()
      _scale_scatter(u, rows)
    return carry

  lax.fori_loop(0, CH // GRP, _group, 0)

  plsc.subcore_barrier()
  pltpu.sync_copy(acc_s.at[pl.ds(s * RPT, RPT)],
                  out_hbm.at[c, pl.ds(s * RPT, RPT)])


def _dis_body(degp_ref, dis_ref, dis2_ref):
  d = degp_ref[0] + degp_ref[1] + 1.0
  inv = lax.rsqrt(d)
  dis = jnp.where(d > 0, inv, jnp.float32(0.0))
  dis_ref[...] = dis
  dis2_ref[...] = dis * dis


_dis_call = pl.pallas_call(
    _dis_body,
    out_shape=(
        jax.ShapeDtypeStruct((NP // D, D), jnp.float32),
        jax.ShapeDtypeStruct((NP // D, D), jnp.float32),
    ),
)


def _mm_body(x_ref, w_ref, dis_ref, h_ref, g_ref):
  h = jnp.dot(x_ref[...], w_ref[...], preferred_element_type=jnp.float32)
  h_ref[...] = h
  g_ref[...] = h * dis_ref[...]


_mm_call = pl.pallas_call(
    _mm_body,
    out_shape=(
        jax.ShapeDtypeStruct((NP, D), jnp.float32),
        jax.ShapeDtypeStruct((NP, D), jnp.float32),
    ),
)


def _comb_body(p_ref, h_ref, dis_ref, dis2_ref, b_ref, z_ref):
  m = (p_ref[0] + p_ref[1]) * dis_ref[...] + h_ref[...] * dis2_ref[...] + b_ref[...]
  z_ref[...] = jnp.maximum(m, 0.0)


_comb_call = pl.pallas_call(
    _comb_body,
    out_shape=jax.ShapeDtypeStruct((NP, D), jnp.float32),
)


def kernel(input, adj, adj_wts, W1, b1, W2, b2):
  x = jnp.pad(input, ((0, NP - N), (0, 0)))
  src = adj[0].astype(jnp.int32)
  dst = adj[1].astype(jnp.int32)
  ew = adj_wts.astype(jnp.float32)
  pad = EP - E
  spread = (jnp.arange(pad, dtype=jnp.int32) * 8) % N
  src_p = jnp.concatenate([src, spread]).reshape(NW, CH, CE)
  dst_p = jnp.concatenate([dst, spread]).reshape(NW, CH, CE)
  ew_p = jnp.concatenate([ew, jnp.zeros((pad,), jnp.float32)]).reshape(NW, CH, CE)

  degp = _deg_kernel(dst_p, ew_p)
  dis80, dis280 = _dis_call(degp.reshape(NC, NP // D, D))
  dis_col = dis80.reshape(NP, 1)
  dis2_col = dis280.reshape(NP, 1)

  h1, g1 = _mm_call(x, W1, dis_col)
  p1 = _agg_kernel(g1, src_p, dst_p, ew_p)
  z1 = _comb_call(p1, h1, dis_col, dis2_col, b1.reshape(1, D))
  h2, g2 = _mm_call(z1, W2, dis_col)
  p2 = _agg_kernel(g2, src_p, dst_p, ew_p)
  z2 = _comb_call(p2, h2, dis_col, dis2_col, b2.reshape(1, D))
  return z2[:N]
